# baseline (speedup 1.0000x reference)
import functools

import jax
import jax.numpy as jnp
from jax import lax
from jax.experimental import pallas as pl
from jax.experimental.pallas import tpu as pltpu
from jax.experimental.pallas import tpu_sc as plsc


def _sc_gather(t_n, ab_tab):
    n = t_n.shape[0]
    info = plsc.get_sparse_core_info()
    lanes = info.num_lanes
    ns = info.num_subcores
    nw = ns
    rpw = n // nw
    rows = rpw // 128
    ngroups = rpw // lanes
    gpr = 128 // lanes

    mesh = plsc.VectorSubcoreMesh(core_axis_name="c", subcore_axis_name="s",
                                  num_cores=1)

    @functools.partial(
        pl.kernel,
        mesh=mesh,
        compiler_params=pltpu.CompilerParams(needs_layout_passes=False),
        out_type=jax.ShapeDtypeStruct((n // 128, 128), jnp.float32),
        scratch_types=[
            pltpu.VMEM((rpw,), jnp.int32),
            pltpu.VMEM(ab_tab.shape, jnp.float32),
            pltpu.VMEM((rows, 128), jnp.float32),
        ],
    )
    def run(t_h, tab_h, out_h, t_v, tab_v, val_v):
        wid = lax.axis_index("s")
        base = wid * rpw
        pltpu.sync_copy(t_h.at[pl.ds(base, rpw)], t_v)
        pltpu.sync_copy(tab_h, tab_v)

        @plsc.parallel_loop(0, ngroups, unroll=4)
        def _group(g):
            tb = t_v[pl.ds(g * lanes, lanes)]
            val_v[g // gpr, pl.ds((g % gpr) * lanes, lanes)] = (
                plsc.load_gather(tab_v, [tb]))

        pltpu.sync_copy(val_v, out_h.at[pl.ds(wid * rows, rows)])

    return run(t_n, ab_tab)


def _sc_copy(noise):
    n, k = noise.shape
    info = plsc.get_sparse_core_info()
    nc, ns = info.num_cores, info.num_subcores
    nw = nc * ns
    rpw = n // nw

    rc = rpw // 2

    mesh = plsc.VectorSubcoreMesh(core_axis_name="c", subcore_axis_name="s")

    @functools.partial(
        pl.kernel,
        mesh=mesh,
        compiler_params=pltpu.CompilerParams(needs_layout_passes=False),
        out_type=jax.ShapeDtypeStruct((n, k), jnp.float32),
        scratch_types=[
            pltpu.VMEM((rc, k), jnp.float32),
            pltpu.VMEM((rc, k), jnp.float32),
            pltpu.SemaphoreType.DMA,
            pltpu.SemaphoreType.DMA,
            pltpu.SemaphoreType.DMA,
            pltpu.SemaphoreType.DMA,
        ],
    )
    def run(nz_h, out_h, b0, b1, si0, si1, so0, so1):
        wid = lax.axis_index("s") * nc + lax.axis_index("c")
        base = wid * rpw
        h0 = pltpu.async_copy(nz_h.at[pl.ds(base, rc)], b0, si0)
        h1 = pltpu.async_copy(nz_h.at[pl.ds(base + rc, rc)], b1, si1)
        h0.wait()
        o0 = pltpu.async_copy(b0, out_h.at[pl.ds(base, rc)], so0)
        h1.wait()
        o1 = pltpu.async_copy(b1, out_h.at[pl.ds(base + rc, rc)], so1)
        o0.wait()
        o1.wait()

    return run(noise)


def _tc_scale_add(ab2, z0, noise):
    n, k = z0.shape
    blk = 4096
    g = blk // 128

    def body(ab_ref, z0_ref, nz_ref, o_ref, nc_ref):
        ab = ab_ref[...]
        a = jnp.sqrt(ab)
        b = jnp.sqrt(jnp.maximum(1.0 - ab, 0.0))
        a_f = lax.broadcast_in_dim(a, (g, 128, k), (0, 1)).reshape(blk, k)
        b_f = lax.broadcast_in_dim(b, (g, 128, k), (0, 1)).reshape(blk, k)
        nz = nz_ref[...]
        o_ref[...] = a_f * z0_ref[...] + b_f * nz
        nc_ref[...] = nz

    return pl.pallas_call(
        body,
        grid=(n // blk,),
        in_specs=[
            pl.BlockSpec((g, 128), lambda i: (i, 0)),
            pl.BlockSpec((blk, k), lambda i: (i, 0)),
            pl.BlockSpec((blk, k), lambda i: (i, 0)),
        ],
        out_specs=[pl.BlockSpec((blk, k), lambda i: (i, 0)),
                   pl.BlockSpec((blk, k), lambda i: (i, 0))],
        out_shape=[jax.ShapeDtypeStruct((n, k), jnp.float32),
                   jax.ShapeDtypeStruct((n, k), jnp.float32)],
    )(ab2, z0, noise)


def kernel(z0_nk, t_n, noise, alpha_bar):
    ab2 = _sc_gather(t_n.astype(jnp.int32), alpha_bar.astype(jnp.float32))
    z_t, noise_out = _tc_scale_add(ab2, z0_nk, noise)
    return (z_t, noise_out)


# speedup vs baseline: 1.0654x; 1.0654x over previous
import functools

import jax
import jax.numpy as jnp
from jax import lax
from jax.experimental import pallas as pl
from jax.experimental.pallas import tpu as pltpu
from jax.experimental.pallas import tpu_sc as plsc


def _sc_gather(t_n, ab_tab):
    n = t_n.shape[0]
    info = plsc.get_sparse_core_info()
    lanes = info.num_lanes
    ns = info.num_subcores
    nw = ns
    rpw = n // nw
    rows = rpw // 128
    ngroups = rpw // lanes
    gpr = 128 // lanes

    mesh = plsc.VectorSubcoreMesh(core_axis_name="c", subcore_axis_name="s",
                                  num_cores=1)

    @functools.partial(
        pl.kernel,
        mesh=mesh,
        compiler_params=pltpu.CompilerParams(needs_layout_passes=False),
        out_type=jax.ShapeDtypeStruct((n // 128, 128), jnp.float32),
        scratch_types=[
            pltpu.VMEM((rpw,), jnp.int32),
            pltpu.VMEM(ab_tab.shape, jnp.float32),
            pltpu.VMEM((rows, 128), jnp.float32),
            pltpu.SemaphoreType.DMA,
            pltpu.SemaphoreType.DMA,
        ],
    )
    def run(t_h, tab_h, out_h, t_v, tab_v, val_v, s0, s1):
        wid = lax.axis_index("s")
        base = wid * rpw
        h0 = pltpu.async_copy(t_h.at[pl.ds(base, rpw)], t_v, s0)
        h1 = pltpu.async_copy(tab_h, tab_v, s1)
        h0.wait()
        h1.wait()

        @plsc.parallel_loop(0, ngroups, unroll=4)
        def _group(g):
            tb = t_v[pl.ds(g * lanes, lanes)]
            val_v[g // gpr, pl.ds((g % gpr) * lanes, lanes)] = (
                plsc.load_gather(tab_v, [tb]))

        pltpu.sync_copy(val_v, out_h.at[pl.ds(wid * rows, rows)])

    return run(t_n, ab_tab)


def _tc_scale_add(ab2, z0, noise):
    n, k = z0.shape
    blk = 8192
    g = blk // 128

    def body(ab_ref, z0_ref, nz_ref, o_ref, nc_ref):
        ab = ab_ref[...]
        a = jnp.sqrt(ab)
        b = jnp.sqrt(jnp.maximum(1.0 - ab, 0.0))
        a_f = lax.broadcast_in_dim(a, (g, 128, k), (0, 1)).reshape(blk, k)
        b_f = lax.broadcast_in_dim(b, (g, 128, k), (0, 1)).reshape(blk, k)
        nz = nz_ref[...]
        o_ref[...] = a_f * z0_ref[...] + b_f * nz
        nc_ref[...] = nz

    return pl.pallas_call(
        body,
        grid=(n // blk,),
        in_specs=[
            pl.BlockSpec((g, 128), lambda i: (i, 0)),
            pl.BlockSpec((blk, k), lambda i: (i, 0)),
            pl.BlockSpec((blk, k), lambda i: (i, 0)),
        ],
        out_specs=[pl.BlockSpec((blk, k), lambda i: (i, 0)),
                   pl.BlockSpec((blk, k), lambda i: (i, 0))],
        out_shape=[jax.ShapeDtypeStruct((n, k), jnp.float32),
                   jax.ShapeDtypeStruct((n, k), jnp.float32)],
    )(ab2, z0, noise)


def kernel(z0_nk, t_n, noise, alpha_bar):
    ab2 = _sc_gather(t_n.astype(jnp.int32), alpha_bar.astype(jnp.float32))
    z_t, noise_out = _tc_scale_add(ab2, z0_nk, noise)
    return (z_t, noise_out)
